# baseline (speedup 1.0000x reference)
import functools

import jax
import jax.numpy as jnp
from jax import lax
from jax.experimental import pallas as pl
from jax.experimental.pallas import tpu as pltpu
from jax.experimental.pallas import tpu_sc as plsc

DIM = 32
NC = 2
NS = 16
NW = NC * NS
LB = 128


@functools.lru_cache(maxsize=None)
def _make_gather(batch: int, width: int, vocab: int):
    nb = batch // LB
    units = width * nb
    assert batch % LB == 0 and units % (2 * NW) == 0, (batch, width)
    upw = units // NW
    db = DIM // 8
    mesh = plsc.VectorSubcoreMesh(
        core_axis_name="c", subcore_axis_name="s",
        num_cores=NC, num_subcores=NS,
    )

    R = 4
    assert upw % R == 0

    @functools.partial(
        pl.kernel,
        out_type=jax.ShapeDtypeStruct((width, db, nb, 8, LB), jnp.float32),
        mesh=mesh,
        scratch_types=[
            pltpu.VMEM((upw * LB,), jnp.int32),
            pltpu.VMEM((R, LB, DIM), jnp.float32),
            pltpu.VMEM((2, DIM, LB), jnp.float32),
        ] + [pltpu.SemaphoreType.DMA] * (R + 2),
        compiler_params=pltpu.CompilerParams(
            use_tc_tiling_on_sc=False, needs_layout_passes=False),
    )
    def k(idx_hbm, table_hbm, om_hbm, idx_v, rows_v, ot_v, *sems):
        gsems = sems[:R]
        wsems = sems[R:R + 2]
        wid = lax.axis_index("s") * NC + lax.axis_index("c")
        ubase = wid * upw
        pltpu.sync_copy(idx_hbm.at[pl.ds(ubase * LB, upw * LB)], idx_v)

        iota16 = lax.iota(jnp.int32, 16)
        rvecs = [iota16 + (16 * j) for j in range(LB // 16)]

        def fire(ul, slot):
            pltpu.async_copy(
                table_hbm.at[idx_v.at[pl.ds(ul * LB, LB)]],
                rows_v.at[slot], gsems[slot])

        def drain(ul, slot):
            pltpu.make_async_copy(
                table_hbm.at[idx_v.at[pl.ds(ul * LB, LB)]],
                rows_v.at[slot], gsems[slot]).wait()

        def wb(w, bb, slot):
            for d0 in range(db):
                pltpu.async_copy(
                    ot_v.at[slot, pl.ds(d0 * 8, 8)],
                    om_hbm.at[w, d0, bb], wsems[slot])

        def wb_wait(w, bb, slot):
            for d0 in range(db):
                pltpu.make_async_copy(
                    ot_v.at[slot, pl.ds(d0 * 8, 8)],
                    om_hbm.at[w, d0, bb], wsems[slot]).wait()

        def transpose(slot, oslot):
            rows = rows_v.at[slot]
            for d in range(DIM):
                cvec = jnp.full((16,), d, jnp.int32)
                for j in range(LB // 16):
                    v = plsc.load_gather(rows, [rvecs[j], cvec])
                    ot_v[oslot, d, pl.ds(16 * j, 16)] = v

        for j in range(R - 1):
            fire(j, j)

        @pl.loop(0, upw, step=R)
        def _ring(u0):
            for b in range(R):
                ul = u0 + b
                u = ubase + ul
                w = lax.div(u, nb)
                bb = lax.rem(u, nb)
                oslot = b % 2

                @pl.when(ul + (R - 1) < upw)
                def _():
                    fire(ul + (R - 1), (b + R - 1) % R)

                drain(ul, b)

                @pl.when(ul >= 2)
                def _():
                    wb_wait(w, bb, oslot)

                transpose(b, oslot)
                wb(w, bb, oslot)

        wb_wait(0, 0, 0)
        wb_wait(0, 0, 1)

    return k


def kernel(input, table):
    batch, width = input.shape
    vocab, dim = table.shape
    assert dim == DIM
    idx_wm = input.T.reshape(batch * width).astype(jnp.int32)
    om = _make_gather(batch, width, vocab)(idx_wm, table)
    out = jnp.transpose(om, (2, 4, 0, 1, 3)).reshape(batch, width, dim)
    return out


# speedup vs baseline: 1.1174x; 1.1174x over previous
import functools

import jax
import jax.numpy as jnp
from jax import lax
from jax.experimental import pallas as pl
from jax.experimental.pallas import tpu as pltpu
from jax.experimental.pallas import tpu_sc as plsc

DIM = 32
NC = 2
NS = 16
NW = NC * NS
LB = 128


@functools.lru_cache(maxsize=None)
def _make_gather(batch: int, width: int, vocab: int):
    nb = batch // LB
    units = width * nb
    assert batch % LB == 0 and units % (2 * NW) == 0, (batch, width)
    upw = units // NW
    db = DIM // 8
    mesh = plsc.VectorSubcoreMesh(
        core_axis_name="c", subcore_axis_name="s",
        num_cores=NC, num_subcores=NS,
    )

    R = 4
    assert upw % R == 0

    @functools.partial(
        pl.kernel,
        out_type=jax.ShapeDtypeStruct((width, db, nb, 8, LB), jnp.float32),
        mesh=mesh,
        scratch_types=[
            pltpu.VMEM((upw * LB,), jnp.int32),
            pltpu.VMEM((R, LB, DIM), jnp.float32),
            pltpu.VMEM((2, DIM, LB), jnp.float32),
        ] + [pltpu.SemaphoreType.DMA] * (R + 2),
        compiler_params=pltpu.CompilerParams(
            use_tc_tiling_on_sc=False, needs_layout_passes=False),
    )
    def k(idx_hbm, table_hbm, om_hbm, idx_v, rows_v, ot_v, *sems):
        gsems = sems[:R]
        wsems = sems[R:R + 2]
        wid = lax.axis_index("s") * NC + lax.axis_index("c")
        ubase = wid * upw
        pltpu.sync_copy(idx_hbm.at[pl.ds(ubase * LB, upw * LB)], idx_v)

        iota16 = lax.iota(jnp.int32, 16)
        rvecs = [iota16 + (16 * j) for j in range(LB // 16)]

        def fire(ul, slot):
            pltpu.async_copy(
                table_hbm.at[idx_v.at[pl.ds(ul * LB, LB)]],
                rows_v.at[slot], gsems[slot])

        def drain(ul, slot):
            pltpu.make_async_copy(
                table_hbm.at[idx_v.at[pl.ds(ul * LB, LB)]],
                rows_v.at[slot], gsems[slot]).wait()

        def wb(w, bb, slot):
            for d0 in range(db):
                pltpu.async_copy(
                    ot_v.at[slot, pl.ds(d0 * 8, 8)],
                    om_hbm.at[w, d0, bb], wsems[slot])

        def wb_wait(w, bb, slot):
            for d0 in range(db):
                pltpu.make_async_copy(
                    ot_v.at[slot, pl.ds(d0 * 8, 8)],
                    om_hbm.at[w, d0, bb], wsems[slot]).wait()

        def transpose(slot, oslot):
            rows = rows_v.at[slot]
            for d0 in range(0, DIM, 2):
                vs = []
                for d in (d0, d0 + 1):
                    cvec = jnp.full((16,), d, jnp.int32)
                    for j in range(LB // 16):
                        vs.append(plsc.load_gather(rows, [rvecs[j], cvec]))
                for i, d in enumerate((d0, d0 + 1)):
                    for j in range(LB // 16):
                        ot_v[oslot, d, pl.ds(16 * j, 16)] = vs[i * (LB // 16) + j]

        for j in range(R - 1):
            fire(j, j)

        @pl.loop(0, upw, step=R)
        def _ring(u0):
            for b in range(R):
                ul = u0 + b
                u = ubase + ul
                w = lax.div(u, nb)
                bb = lax.rem(u, nb)
                oslot = b % 2

                @pl.when(ul + (R - 1) < upw)
                def _():
                    fire(ul + (R - 1), (b + R - 1) % R)

                drain(ul, b)

                @pl.when(ul >= 2)
                def _():
                    wb_wait(w, bb, oslot)

                transpose(b, oslot)
                wb(w, bb, oslot)

        wb_wait(0, 0, 0)
        wb_wait(0, 0, 1)

    return k


def kernel(input, table):
    batch, width = input.shape
    vocab, dim = table.shape
    assert dim == DIM
    idx_wm = input.T.reshape(batch * width).astype(jnp.int32)
    om = _make_gather(batch, width, vocab)(idx_wm, table)
    out = jnp.transpose(om, (2, 4, 0, 1, 3)).reshape(batch, width, dim)
    return out


# speedup vs baseline: 1.3852x; 1.2397x over previous
import functools

import jax
import jax.numpy as jnp
from jax import lax
from jax.experimental import pallas as pl
from jax.experimental.pallas import tpu as pltpu
from jax.experimental.pallas import tpu_sc as plsc

DIM = 32
NC = 2
NS = 16
NW = NC * NS
LB = 128


@functools.lru_cache(maxsize=None)
def _make_gather(batch: int, width: int, vocab: int):
    nb = batch // LB
    units = width * nb
    assert batch % LB == 0 and units % (2 * NW) == 0, (batch, width)
    upw = units // NW
    db = DIM // 8
    mesh = plsc.VectorSubcoreMesh(
        core_axis_name="c", subcore_axis_name="s",
        num_cores=NC, num_subcores=NS,
    )

    R = 4
    assert upw % R == 0

    @functools.partial(
        pl.kernel,
        out_type=jax.ShapeDtypeStruct((width, db, nb, 8, LB), jnp.float32),
        mesh=mesh,
        scratch_types=[
            pltpu.VMEM((upw * LB,), jnp.int32),
            pltpu.VMEM((R, LB, DIM), jnp.float32),
            pltpu.VMEM((2, DIM, LB + 1), jnp.float32),
        ] + [pltpu.SemaphoreType.DMA] * (R + 2),
        compiler_params=pltpu.CompilerParams(
            use_tc_tiling_on_sc=False, needs_layout_passes=False),
    )
    def k(idx_hbm, table_hbm, om_hbm, idx_v, rows_v, ot_v, *sems):
        gsems = sems[:R]
        wsems = sems[R:R + 2]
        wid = lax.axis_index("s") * NC + lax.axis_index("c")
        ubase = wid * upw
        pltpu.sync_copy(idx_hbm.at[pl.ds(ubase * LB, upw * LB)], idx_v)

        iota16 = lax.iota(jnp.int32, 16)
        dvecs = [iota16 + (16 * h) for h in range(DIM // 16)]

        def fire(ul, slot):
            pltpu.async_copy(
                table_hbm.at[idx_v.at[pl.ds(ul * LB, LB)]],
                rows_v.at[slot], gsems[slot])

        def drain(ul, slot):
            pltpu.make_async_copy(
                table_hbm.at[idx_v.at[pl.ds(ul * LB, LB)]],
                rows_v.at[slot], gsems[slot]).wait()

        def wb(w, bb, slot):
            for d0 in range(db):
                pltpu.async_copy(
                    ot_v.at[slot, pl.ds(d0 * 8, 8), pl.ds(0, LB)],
                    om_hbm.at[w, d0, bb], wsems[slot])

        def wb_wait(w, bb, slot):
            for d0 in range(db):
                pltpu.make_async_copy(
                    ot_v.at[slot, pl.ds(d0 * 8, 8), pl.ds(0, LB)],
                    om_hbm.at[w, d0, bb], wsems[slot]).wait()

        def transpose(slot, oslot):
            ot = ot_v.at[oslot]
            nh = DIM // 16
            for l0 in range(0, LB, 8):
                vs = [rows_v[slot, l0 + i, pl.ds(16 * h, 16)]
                      for i in range(8) for h in range(nh)]
                for i in range(8):
                    lvec = jnp.full((16,), l0 + i, jnp.int32)
                    for h in range(nh):
                        plsc.store_scatter(ot, [dvecs[h], lvec], vs[i * nh + h])

        for j in range(R - 1):
            fire(j, j)

        @pl.loop(0, upw, step=R)
        def _ring(u0):
            for b in range(R):
                ul = u0 + b
                u = ubase + ul
                w = lax.div(u, nb)
                bb = lax.rem(u, nb)
                oslot = b % 2

                @pl.when(ul + (R - 1) < upw)
                def _():
                    fire(ul + (R - 1), (b + R - 1) % R)

                drain(ul, b)

                @pl.when(ul >= 2)
                def _():
                    wb_wait(w, bb, oslot)

                transpose(b, oslot)
                wb(w, bb, oslot)

        wb_wait(0, 0, 0)
        wb_wait(0, 0, 1)

    return k


def kernel(input, table):
    batch, width = input.shape
    vocab, dim = table.shape
    assert dim == DIM
    idx_wm = input.T.reshape(batch * width).astype(jnp.int32)
    om = _make_gather(batch, width, vocab)(idx_wm, table)
    out = jnp.transpose(om, (2, 4, 0, 1, 3)).reshape(batch, width, dim)
    return out
